# baseline (speedup 1.0000x reference)
import functools

import jax
import jax.numpy as jnp
from jax import lax
from jax.experimental import pallas as pl
from jax.experimental.pallas import tpu as pltpu
from jax.experimental.pallas import tpu_sc as plsc

NUM_CORES = 2
NUM_SUBCORES = 16
NUM_WORKERS = NUM_CORES * NUM_SUBCORES
LANES = 16

BATCH = 16384
EMBED_DIM = 64
NUM_ENT = 1000000
NUM_REL = 1000

NCHUNK = 62
WCAP = 1024
SCAP = 176
CCAP = 64
SLOTS = 2048
DUMMY_E = 1 << 20
DUMP_ROW = BATCH

STRIP = 4096


def _route_body(head_hbm, tail_hbm, etab_t, etab_last,
                hrows_hbm, trows_hbm, bposh_hbm, bpost_hbm,
                strip, wl_he, wl_hb, wl_te, wl_tb,
                sup_he, sup_hb, sup_te, sup_tb,
                cl_he0, cl_hb0, cl_te0, cl_tb0,
                cl_he1, cl_hb1, cl_te1, cl_tb1,
                bposh_v, bpost_v,
                ebuf0, ebuf1, obuf0, obuf1,
                stsem0, stsem1, ssem0, ssem1):
  w = lax.axis_index("s") * NUM_CORES + lax.axis_index("c")
  lanes = lax.iota(jnp.int32, LANES)

  def strip_filter(x_hbm, le, lb):
    pos = 0
    for st in range(BATCH // STRIP):
      pltpu.sync_copy(x_hbm.at[pl.ds(st * STRIP, STRIP)], strip)

      def fbody(k, p):
        e = strip[pl.ds(k * LANES, LANES)]
        m = ((e >> 7) & 31) == w
        b = (st * STRIP + k * LANES) + lanes
        plsc.store_compressed(le.at[pl.ds(p, LANES)], e, mask=m)
        plsc.store_compressed(lb.at[pl.ds(p, LANES)], b, mask=m)
        return p + plsc.all_reduce_population_count(m)[0]

      pos = lax.fori_loop(0, STRIP // LANES, fbody, pos, unroll=2)
    return pos

  nh = strip_filter(head_hbm, wl_he, wl_hb)
  nt = strip_filter(tail_hbm, wl_te, wl_tb)

  dummy_vec = jnp.full((LANES,), DUMMY_E, jnp.int32)
  for sup_x in (sup_he, sup_te):
    def pre(i, carry, sup_x=sup_x):
      sup_x[pl.ds(i * LANES, LANES)] = dummy_vec
      return carry
    lax.fori_loop(0, 8 * SCAP // LANES, pre, 0)

  def sup_split(le, lb, n, sup_e, sup_b):
    for s in range(8):
      def sbody(i, p, s=s):
        e = le[pl.ds(i * LANES, LANES)]
        valid = (i * LANES + lanes) < n
        m = ((e >> 17) == s) & valid
        b = lb[pl.ds(i * LANES, LANES)]
        plsc.store_compressed(sup_e.at[pl.ds(s * SCAP + p, LANES)], e,
                              mask=m)
        plsc.store_compressed(sup_b.at[pl.ds(s * SCAP + p, LANES)], b,
                              mask=m)
        return p + plsc.all_reduce_population_count(m)[0]

      ps = lax.fori_loop(0, WCAP // LANES, sbody, 0, unroll=2)
      sup_e[pl.ds(s * SCAP + ps, LANES)] = dummy_vec

  sup_split(wl_he, wl_hb, nh, sup_he, sup_hb)
  sup_split(wl_te, wl_tb, nt, sup_te, sup_tb)

  def stage_io(c, ebuf, stsem, start):
    def go(cp):
      cp.start() if start else cp.wait()

    @pl.when(c < NCHUNK - 1)
    def _():
      for q in range(4):
        go(pltpu.make_async_copy(
            etab_t.at[:, pl.ds(w * 128 + c * 16384 + q * 4096, 128)],
            ebuf.at[:, pl.ds(q * 128, 128)], stsem))

    @pl.when(c == NCHUNK - 1)
    def _():
      @pl.when(w < 4)
      def _():
        go(pltpu.make_async_copy(
            etab_t.at[:, pl.ds(w * 128 + 999424, 128)],
            ebuf.at[:, pl.ds(0, 128)], stsem))

      @pl.when(w == 4)
      def _():
        go(pltpu.make_async_copy(etab_last, ebuf.at[:, pl.ds(0, 128)],
                                 stsem))

  def chunk_filter(c, sup_e, sup_b, cl_e, cl_b):
    sbase = (c >> 3) * SCAP

    def cbody(i, p):
      e = sup_e[pl.ds(sbase + i * LANES, LANES)]
      m = (e >> 14) == c
      b = sup_b[pl.ds(sbase + i * LANES, LANES)]
      plsc.store_compressed(cl_e.at[pl.ds(p, LANES)], e, mask=m)
      plsc.store_compressed(cl_b.at[pl.ds(p, LANES)], b, mask=m)
      return p + plsc.all_reduce_population_count(m)[0]

    n = lax.fori_loop(0, SCAP // LANES, cbody, 0, unroll=2)
    cl_e[pl.ds(n, LANES)] = jnp.full((LANES,), w << 7, jnp.int32)
    cl_b[pl.ds(n, LANES)] = jnp.full((LANES,), DUMP_ROW, jnp.int32)
    return n

  def extract(ebuf, obuf, cl_e, cl_b, n, rowbase, off, dst, bpos_v, ssem):
    for g in range(3):
      @pl.when(n > g * LANES)
      def _(g=g):
        ev = cl_e[pl.ds(g * LANES, LANES)]
        col = ((ev >> 12) & 3) * 128 + (ev & 127)
        rows = lanes + (rowbase + g * LANES)

        def dbody(d, carry):
          dv = jnp.full((LANES,), d, jnp.int32)
          vals = plsc.load_gather(ebuf, [dv, col])
          plsc.store_scatter(obuf, [rows, dv], vals)
          return carry

        lax.fori_loop(0, EMBED_DIM, dbody, 0, unroll=8)
        woff = pl.multiple_of(w * SLOTS + off + g * LANES, 8)
        pltpu.make_async_copy(
            obuf.at[pl.ds(rowbase + g * LANES, LANES)],
            dst.at[pl.ds(woff, LANES)],
            ssem).start()
        bpos_v[pl.ds(off + g * LANES, LANES)] = cl_b[pl.ds(g * LANES,
                                                           LANES)]
    return off + (jnp.where(n > 0, LANES, 0)
                  + jnp.where(n > LANES, LANES, 0)
                  + jnp.where(n > 2 * LANES, LANES, 0))

  def drain(n, dst, ssem):
    for g in range(3):
      @pl.when(n > g * LANES)
      def _(g=g):
        pltpu.make_async_copy(
            obuf0.at[pl.ds(0, LANES)],
            dst.at[pl.ds(pl.multiple_of(w * SLOTS, 8), LANES)],
            ssem).wait()

  def arm(c, ebuf, obuf, cl_he, cl_hb, cl_te, cl_tb, stsem, ssem,
          nh_prev, nt_prev, offh, offt):
    stage_io(c, ebuf, stsem, start=False)
    drain(nh_prev, hrows_hbm, ssem)
    drain(nt_prev, trows_hbm, ssem)
    nhc = chunk_filter(c, sup_he, sup_hb, cl_he, cl_hb)
    ntc = chunk_filter(c, sup_te, sup_tb, cl_te, cl_tb)
    offh = extract(ebuf, obuf, cl_he, cl_hb, nhc, 0, offh, hrows_hbm,
                   bposh_v, ssem)
    offt = extract(ebuf, obuf, cl_te, cl_tb, ntc, 48, offt, trows_hbm,
                   bpost_v, ssem)

    @pl.when(c + 2 < NCHUNK)
    def _():
      stage_io(c + 2, ebuf, stsem, start=True)

    return nhc, ntc, offh, offt

  stage_io(0, ebuf0, stsem0, start=True)
  stage_io(1, ebuf1, stsem1, start=True)

  def loop_body(cc, carry):
    nh0, nt0, nh1, nt1, offh, offt = carry
    nh0, nt0, offh, offt = arm(2 * cc, ebuf0, obuf0, cl_he0, cl_hb0,
                               cl_te0, cl_tb0, stsem0, ssem0,
                               nh0, nt0, offh, offt)
    nh1, nt1, offh, offt = arm(2 * cc + 1, ebuf1, obuf1, cl_he1, cl_hb1,
                               cl_te1, cl_tb1, stsem1, ssem1,
                               nh1, nt1, offh, offt)
    return nh0, nt0, nh1, nt1, offh, offt

  z = jnp.int32(0)
  nh0, nt0, nh1, nt1, offh, offt = lax.fori_loop(
      0, 30, loop_body, (z, z, z, z, z, z))
  nh0, nt0, offh, offt = arm(jnp.int32(60), ebuf0, obuf0, cl_he0, cl_hb0,
                             cl_te0, cl_tb0, stsem0, ssem0,
                             nh0, nt0, offh, offt)
  nh1, nt1, offh, offt = arm(jnp.int32(61), ebuf1, obuf1, cl_he1, cl_hb1,
                             cl_te1, cl_tb1, stsem1, ssem1,
                             nh1, nt1, offh, offt)
  drain(nh0, hrows_hbm, ssem0)
  drain(nt0, trows_hbm, ssem0)
  drain(nh1, hrows_hbm, ssem1)
  drain(nt1, trows_hbm, ssem1)

  dump_vec = jnp.full((LANES,), DUMP_ROW, jnp.int32)

  def padmap(i, carry):
    sl = pl.ds(i * LANES, LANES)
    posv = i * LANES + lanes
    bh = bposh_v[sl]
    bt = bpost_v[sl]
    bposh_v[sl] = jnp.where(posv >= offh, dump_vec, bh)
    bpost_v[sl] = jnp.where(posv >= offt, dump_vec, bt)
    return carry

  lax.fori_loop(0, SLOTS // LANES, padmap, 0, unroll=4)
  pltpu.sync_copy(bposh_v, bposh_hbm.at[pl.ds(w * SLOTS, SLOTS)])
  pltpu.sync_copy(bpost_v, bpost_hbm.at[pl.ds(w * SLOTS, SLOTS)])


def _compute_body(rel_hbm, hrows_hbm, trows_hbm, bposh_hbm, bpost_hbm,
                  rtab, out_t,
                  ridx, rtb, sloth, slott, bstrip,
                  hbuf0, tbuf0, hbuf1, tbuf1, obuf,
                  gsem0, gsem1, *, rows_per_worker):
  w = lax.axis_index("s") * NUM_CORES + lax.axis_index("c")
  base = w * rows_per_worker
  lanes = lax.iota(jnp.int32, LANES)

  pltpu.sync_copy(rel_hbm.at[pl.ds(base, rows_per_worker)], ridx)
  pltpu.sync_copy(rtab, rtb)

  total = NUM_WORKERS * SLOTS
  for bmap, slotref in ((bposh_hbm, sloth), (bpost_hbm, slott)):
    for st in range(total // STRIP):
      pltpu.sync_copy(bmap.at[pl.ds(st * STRIP, STRIP)], bstrip)

      def ibody(k, carry, st=st, slotref=slotref):
        b = bstrip[pl.ds(k * LANES, LANES)]
        slotv = (st * STRIP + k * LANES) + lanes
        m = (b >= base) & (b < base + rows_per_worker)
        idx = (b - base) & (rows_per_worker - 1)
        plsc.store_scatter(slotref, [idx], slotv, mask=m)
        return carry

      lax.fori_loop(0, STRIP // LANES, ibody, 0, unroll=4)

  GR = 64

  def gstage(g, hbuf, tbuf, gsem, start):
    def go(cp):
      cp.start() if start else cp.wait()
    go(pltpu.make_async_copy(hrows_hbm.at[sloth.at[pl.ds(g * GR, GR)]],
                             hbuf, gsem))
    go(pltpu.make_async_copy(trows_hbm.at[slott.at[pl.ds(g * GR, GR)]],
                             tbuf, gsem))

  ngroup = rows_per_worker // GR

  def garm(g, hbuf, tbuf, gsem):
    gstage(g, hbuf, tbuf, gsem, start=False)
    for sg in range(GR // LANES):
      rows = lanes + sg * LANES
      roff = ridx[pl.ds(g * GR + sg * LANES, LANES)]

      def dotb(d, acc):
        dv = jnp.full((LANES,), d, jnp.int32)
        hv = plsc.load_gather(hbuf, [rows, dv])
        tv = plsc.load_gather(tbuf, [rows, dv])
        return acc + hv * tv

      corr = lax.fori_loop(0, EMBED_DIM, dotb,
                           jnp.zeros((LANES,), jnp.float32), unroll=8)

      def outb(d, carry):
        dv = jnp.full((LANES,), d, jnp.int32)
        rv = plsc.load_gather(rtb, [dv, roff])
        x = corr * rv
        obuf[d, pl.ds((g % 4) * GR + sg * LANES, LANES)] = (
            1.0 / (1.0 + jnp.exp(-x)))
        return carry

      lax.fori_loop(0, EMBED_DIM, outb, 0, unroll=8)

    @pl.when(g + 2 < ngroup)
    def _():
      gstage(g + 2, hbuf, tbuf, gsem, start=True)

    @pl.when(g % 4 == 3)
    def _():
      fb = pl.multiple_of(base + (g - 3) * GR, 128)
      pltpu.sync_copy(obuf, out_t.at[:, pl.ds(fb, 4 * GR)])

  gstage(0, hbuf0, tbuf0, gsem0, start=True)
  gstage(1, hbuf1, tbuf1, gsem1, start=True)

  def gloop(gg, carry):
    garm(2 * gg, hbuf0, tbuf0, gsem0)
    garm(2 * gg + 1, hbuf1, tbuf1, gsem1)
    return carry

  lax.fori_loop(0, ngroup // 2, gloop, 0)


def _build():
  mesh = plsc.VectorSubcoreMesh(core_axis_name="c", subcore_axis_name="s",
                                num_cores=NUM_CORES,
                                num_subcores=NUM_SUBCORES)
  i32, f32 = jnp.int32, jnp.float32
  route = pl.kernel(
      _route_body,
      out_type=(jax.ShapeDtypeStruct((NUM_WORKERS * SLOTS, 128), f32),
                jax.ShapeDtypeStruct((NUM_WORKERS * SLOTS, 128), f32),
                jax.ShapeDtypeStruct((NUM_WORKERS * SLOTS,), i32),
                jax.ShapeDtypeStruct((NUM_WORKERS * SLOTS,), i32)),
      mesh=mesh,
      scratch_types=[
          pltpu.VMEM((STRIP,), i32),
          pltpu.VMEM((WCAP,), i32), pltpu.VMEM((WCAP,), i32),
          pltpu.VMEM((WCAP,), i32), pltpu.VMEM((WCAP,), i32),
          pltpu.VMEM((8 * SCAP,), i32), pltpu.VMEM((8 * SCAP,), i32),
          pltpu.VMEM((8 * SCAP,), i32), pltpu.VMEM((8 * SCAP,), i32),
          pltpu.VMEM((CCAP,), i32), pltpu.VMEM((CCAP,), i32),
          pltpu.VMEM((CCAP,), i32), pltpu.VMEM((CCAP,), i32),
          pltpu.VMEM((CCAP,), i32), pltpu.VMEM((CCAP,), i32),
          pltpu.VMEM((CCAP,), i32), pltpu.VMEM((CCAP,), i32),
          pltpu.VMEM((SLOTS,), i32), pltpu.VMEM((SLOTS,), i32),
          pltpu.VMEM((EMBED_DIM, 512), f32),
          pltpu.VMEM((EMBED_DIM, 512), f32),
          pltpu.VMEM((96, 128), f32),
          pltpu.VMEM((96, 128), f32),
          pltpu.SemaphoreType.DMA, pltpu.SemaphoreType.DMA,
          pltpu.SemaphoreType.DMA, pltpu.SemaphoreType.DMA,
      ],
      compiler_params=pltpu.CompilerParams(needs_layout_passes=False),
  )

  rows_per_worker = BATCH // NUM_WORKERS
  compute = pl.kernel(
      functools.partial(_compute_body, rows_per_worker=rows_per_worker),
      out_type=jax.ShapeDtypeStruct((EMBED_DIM, BATCH), f32),
      mesh=mesh,
      scratch_types=[
          pltpu.VMEM((rows_per_worker,), i32),
          pltpu.VMEM((EMBED_DIM, 1024), f32),
          pltpu.VMEM((rows_per_worker,), i32),
          pltpu.VMEM((rows_per_worker,), i32),
          pltpu.VMEM((STRIP,), i32),
          pltpu.VMEM((64, 128), f32), pltpu.VMEM((64, 128), f32),
          pltpu.VMEM((64, 128), f32), pltpu.VMEM((64, 128), f32),
          pltpu.VMEM((EMBED_DIM, 256), f32),
          pltpu.SemaphoreType.DMA, pltpu.SemaphoreType.DMA,
      ],
      compiler_params=pltpu.CompilerParams(needs_layout_passes=False),
  )
  return route, compute


_route, _compute = _build()


def kernel(head, relation, tail, entity_table, relation_table):
  head = head.astype(jnp.int32)
  relation = relation.astype(jnp.int32)
  tail = tail.astype(jnp.int32)
  etab_t = entity_table.T
  etab_last = jnp.pad(entity_table[999936:], ((0, 64), (0, 0))).T
  rtab = jnp.pad(relation_table, ((0, 1024 - NUM_REL), (0, 0))).T
  hrows, trows, bposh, bpost = _route(head, tail, etab_t, etab_last)
  out_t = _compute(relation, hrows, trows, bposh, bpost, rtab)
  return out_t.T


# speedup vs baseline: 1.0558x; 1.0558x over previous
import functools

import jax
import jax.numpy as jnp
from jax import lax
from jax.experimental import pallas as pl
from jax.experimental.pallas import tpu as pltpu
from jax.experimental.pallas import tpu_sc as plsc

NUM_CORES = 2
NUM_SUBCORES = 16
NUM_WORKERS = NUM_CORES * NUM_SUBCORES
LANES = 16

BATCH = 16384
EMBED_DIM = 64
NUM_ENT = 1000000
NUM_REL = 1000

NCHUNK = 62
WCAP = 1024
SCAP = 176
CCAP = 64
SLOTS = 2048
DUMMY_E = 1 << 20
DUMP_ROW = BATCH

STRIP = 4096


def _route_body(head_hbm, tail_hbm, etab_t, etab_last,
                hrows_hbm, trows_hbm, bposh_hbm, bpost_hbm,
                strip, wl_he, wl_hb, wl_te, wl_tb,
                sup_he, sup_hb, sup_te, sup_tb,
                cl_he0, cl_hb0, cl_te0, cl_tb0,
                cl_he1, cl_hb1, cl_te1, cl_tb1,
                bposh_v, bpost_v,
                ebuf0, ebuf1, obuf0, obuf1,
                stsem0, stsem1, ssem0, ssem1):
  w = lax.axis_index("s") * NUM_CORES + lax.axis_index("c")
  lanes = lax.iota(jnp.int32, LANES)

  def strip_filter(x_hbm, le, lb):
    pos = 0
    for st in range(BATCH // STRIP):
      pltpu.sync_copy(x_hbm.at[pl.ds(st * STRIP, STRIP)], strip)

      def fbody(k, p):
        e = strip[pl.ds(k * LANES, LANES)]
        m = ((e >> 7) & 31) == w
        b = (st * STRIP + k * LANES) + lanes
        plsc.store_compressed(le.at[pl.ds(p, LANES)], e, mask=m)
        plsc.store_compressed(lb.at[pl.ds(p, LANES)], b, mask=m)
        return p + plsc.all_reduce_population_count(m)[0]

      pos = lax.fori_loop(0, STRIP // LANES, fbody, pos, unroll=2)
    return pos

  nh = strip_filter(head_hbm, wl_he, wl_hb)
  nt = strip_filter(tail_hbm, wl_te, wl_tb)

  dummy_vec = jnp.full((LANES,), DUMMY_E, jnp.int32)
  for sup_x in (sup_he, sup_te):
    def pre(i, carry, sup_x=sup_x):
      sup_x[pl.ds(i * LANES, LANES)] = dummy_vec
      return carry
    lax.fori_loop(0, 8 * SCAP // LANES, pre, 0)

  def sup_split(le, lb, n, sup_e, sup_b):
    for s in range(8):
      def sbody(i, p, s=s):
        e = le[pl.ds(i * LANES, LANES)]
        valid = (i * LANES + lanes) < n
        m = ((e >> 17) == s) & valid
        b = lb[pl.ds(i * LANES, LANES)]
        plsc.store_compressed(sup_e.at[pl.ds(s * SCAP + p, LANES)], e,
                              mask=m)
        plsc.store_compressed(sup_b.at[pl.ds(s * SCAP + p, LANES)], b,
                              mask=m)
        return p + plsc.all_reduce_population_count(m)[0]

      ps = lax.fori_loop(0, WCAP // LANES, sbody, 0, unroll=2)
      sup_e[pl.ds(s * SCAP + ps, LANES)] = dummy_vec

  sup_split(wl_he, wl_hb, nh, sup_he, sup_hb)
  sup_split(wl_te, wl_tb, nt, sup_te, sup_tb)

  def stage_io(c, ebuf, stsem, start):
    def go(cp):
      cp.start() if start else cp.wait()

    @pl.when(c < NCHUNK - 1)
    def _():
      for q in range(4):
        go(pltpu.make_async_copy(
            etab_t.at[:, pl.ds(w * 128 + c * 16384 + q * 4096, 128)],
            ebuf.at[:, pl.ds(q * 128, 128)], stsem))

    @pl.when(c == NCHUNK - 1)
    def _():
      @pl.when(w < 4)
      def _():
        go(pltpu.make_async_copy(
            etab_t.at[:, pl.ds(w * 128 + 999424, 128)],
            ebuf.at[:, pl.ds(0, 128)], stsem))

      @pl.when(w == 4)
      def _():
        go(pltpu.make_async_copy(etab_last, ebuf.at[:, pl.ds(0, 128)],
                                 stsem))

  def chunk_filter(c, sup_e, sup_b, cl_e, cl_b):
    sbase = (c >> 3) * SCAP

    def cbody(i, p):
      e = sup_e[pl.ds(sbase + i * LANES, LANES)]
      m = (e >> 14) == c
      b = sup_b[pl.ds(sbase + i * LANES, LANES)]
      plsc.store_compressed(cl_e.at[pl.ds(p, LANES)], e, mask=m)
      plsc.store_compressed(cl_b.at[pl.ds(p, LANES)], b, mask=m)
      return p + plsc.all_reduce_population_count(m)[0]

    n = lax.fori_loop(0, SCAP // LANES, cbody, 0, unroll=2)
    cl_e[pl.ds(n, LANES)] = jnp.full((LANES,), w << 7, jnp.int32)
    cl_b[pl.ds(n, LANES)] = jnp.full((LANES,), DUMP_ROW, jnp.int32)
    return n

  def extract(ebuf, obuf, cl_e, cl_b, n, rowbase, off, dst, bpos_v, ssem):
    for g in range(3):
      @pl.when(n > g * LANES)
      def _(g=g):
        ev = cl_e[pl.ds(g * LANES, LANES)]
        col = ((ev >> 12) & 3) * 128 + (ev & 127)
        rows = lanes + (rowbase + g * LANES)

        def dbody(d, carry):
          dv = jnp.full((LANES,), d, jnp.int32)
          vals = plsc.load_gather(ebuf, [dv, col])
          plsc.store_scatter(obuf, [rows, dv], vals)
          return carry

        lax.fori_loop(0, EMBED_DIM, dbody, 0, unroll=8)
        woff = pl.multiple_of(w * SLOTS + off + g * LANES, 8)
        pltpu.make_async_copy(
            obuf.at[pl.ds(rowbase + g * LANES, LANES)],
            dst.at[pl.ds(woff, LANES)],
            ssem).start()
        bpos_v[pl.ds(off + g * LANES, LANES)] = cl_b[pl.ds(g * LANES,
                                                           LANES)]
    return off + (jnp.where(n > 0, LANES, 0)
                  + jnp.where(n > LANES, LANES, 0)
                  + jnp.where(n > 2 * LANES, LANES, 0))

  def drain(n, dst, ssem):
    for g in range(3):
      @pl.when(n > g * LANES)
      def _(g=g):
        pltpu.make_async_copy(
            obuf0.at[pl.ds(0, LANES)],
            dst.at[pl.ds(pl.multiple_of(w * SLOTS, 8), LANES)],
            ssem).wait()

  def arm(c, ebuf, obuf, cl_he, cl_hb, cl_te, cl_tb, stsem, ssem,
          nh_prev, nt_prev, offh, offt):
    stage_io(c, ebuf, stsem, start=False)
    drain(nh_prev, hrows_hbm, ssem)
    drain(nt_prev, trows_hbm, ssem)
    nhc = chunk_filter(c, sup_he, sup_hb, cl_he, cl_hb)
    ntc = chunk_filter(c, sup_te, sup_tb, cl_te, cl_tb)
    offh = extract(ebuf, obuf, cl_he, cl_hb, nhc, 0, offh, hrows_hbm,
                   bposh_v, ssem)
    offt = extract(ebuf, obuf, cl_te, cl_tb, ntc, 48, offt, trows_hbm,
                   bpost_v, ssem)

    @pl.when(c + 2 < NCHUNK)
    def _():
      stage_io(c + 2, ebuf, stsem, start=True)

    return nhc, ntc, offh, offt

  stage_io(0, ebuf0, stsem0, start=True)
  stage_io(1, ebuf1, stsem1, start=True)

  def loop_body(cc, carry):
    nh0, nt0, nh1, nt1, offh, offt = carry
    nh0, nt0, offh, offt = arm(2 * cc, ebuf0, obuf0, cl_he0, cl_hb0,
                               cl_te0, cl_tb0, stsem0, ssem0,
                               nh0, nt0, offh, offt)
    nh1, nt1, offh, offt = arm(2 * cc + 1, ebuf1, obuf1, cl_he1, cl_hb1,
                               cl_te1, cl_tb1, stsem1, ssem1,
                               nh1, nt1, offh, offt)
    return nh0, nt0, nh1, nt1, offh, offt

  z = jnp.int32(0)
  nh0, nt0, nh1, nt1, offh, offt = lax.fori_loop(
      0, 30, loop_body, (z, z, z, z, z, z))
  nh0, nt0, offh, offt = arm(jnp.int32(60), ebuf0, obuf0, cl_he0, cl_hb0,
                             cl_te0, cl_tb0, stsem0, ssem0,
                             nh0, nt0, offh, offt)
  nh1, nt1, offh, offt = arm(jnp.int32(61), ebuf1, obuf1, cl_he1, cl_hb1,
                             cl_te1, cl_tb1, stsem1, ssem1,
                             nh1, nt1, offh, offt)
  drain(nh0, hrows_hbm, ssem0)
  drain(nt0, trows_hbm, ssem0)
  drain(nh1, hrows_hbm, ssem1)
  drain(nt1, trows_hbm, ssem1)

  dump_vec = jnp.full((LANES,), DUMP_ROW, jnp.int32)

  def padmap(i, carry):
    sl = pl.ds(i * LANES, LANES)
    posv = i * LANES + lanes
    bh = bposh_v[sl]
    bt = bpost_v[sl]
    bposh_v[sl] = jnp.where(posv >= offh, dump_vec, bh)
    bpost_v[sl] = jnp.where(posv >= offt, dump_vec, bt)
    return carry

  lax.fori_loop(0, SLOTS // LANES, padmap, 0, unroll=4)
  pltpu.sync_copy(bposh_v, bposh_hbm.at[pl.ds(w * SLOTS, SLOTS)])
  pltpu.sync_copy(bpost_v, bpost_hbm.at[pl.ds(w * SLOTS, SLOTS)])


def _compute_body(rel_hbm, hrows_hbm, trows_hbm, bposh_hbm, bpost_hbm,
                  rtab, out_t,
                  ridx, rtb, sloth, slott, bstrip, bstrip2, bstrip3,
                  bstrip4, hbuf0, tbuf0, hbuf1, tbuf1, obuf,
                  gsem0, gsem1, *, rows_per_worker):
  w = lax.axis_index("s") * NUM_CORES + lax.axis_index("c")
  base = w * rows_per_worker
  lanes = lax.iota(jnp.int32, LANES)

  pltpu.sync_copy(rel_hbm.at[pl.ds(base, rows_per_worker)], ridx)
  pltpu.sync_copy(rtab, rtb)

  total = NUM_WORKERS * SLOTS
  ISTRIP = 2048
  nstrip = total // ISTRIP

  def istage(st, start):
    def go(cp):
      cp.start() if start else cp.wait()
    go(pltpu.make_async_copy(bposh_hbm.at[pl.ds(st * ISTRIP, ISTRIP)],
                             bstrip if st % 2 == 0 else bstrip3, gsem0))
    go(pltpu.make_async_copy(bpost_hbm.at[pl.ds(st * ISTRIP, ISTRIP)],
                             bstrip2 if st % 2 == 0 else bstrip4, gsem1))

  istage(0, True)
  istage(1, True)
  for st in range(nstrip):
    istage(st, False)
    sh = bstrip if st % 2 == 0 else bstrip3
    stt = bstrip2 if st % 2 == 0 else bstrip4

    def ibody(k, carry, st=st, sh=sh, stt=stt):
      slotv = (st * ISTRIP + k * LANES) + lanes
      bh = sh[pl.ds(k * LANES, LANES)]
      mh = (bh >= base) & (bh < base + rows_per_worker)
      plsc.store_scatter(sloth, [(bh - base) & (rows_per_worker - 1)],
                         slotv, mask=mh)
      bt = stt[pl.ds(k * LANES, LANES)]
      mt = (bt >= base) & (bt < base + rows_per_worker)
      plsc.store_scatter(slott, [(bt - base) & (rows_per_worker - 1)],
                         slotv, mask=mt)
      return carry

    lax.fori_loop(0, ISTRIP // LANES, ibody, 0, unroll=4)
    if st + 2 < nstrip:
      istage(st + 2, True)

  GR = 32

  def gstage(g, hbuf, tbuf, gsem, start):
    def go(cp):
      cp.start() if start else cp.wait()
    go(pltpu.make_async_copy(hrows_hbm.at[sloth.at[pl.ds(g * GR, GR)]],
                             hbuf, gsem))
    go(pltpu.make_async_copy(trows_hbm.at[slott.at[pl.ds(g * GR, GR)]],
                             tbuf, gsem))

  ngroup = rows_per_worker // GR

  def garm(g, hbuf, tbuf, gsem):
    gstage(g, hbuf, tbuf, gsem, start=False)
    for sg in range(GR // LANES):
      rows = lanes + sg * LANES
      roff = ridx[pl.ds(g * GR + sg * LANES, LANES)]

      def dotb(d, acc):
        dv = jnp.full((LANES,), d, jnp.int32)
        hv = plsc.load_gather(hbuf, [rows, dv])
        tv = plsc.load_gather(tbuf, [rows, dv])
        return acc + hv * tv

      corr = lax.fori_loop(0, EMBED_DIM, dotb,
                           jnp.zeros((LANES,), jnp.float32), unroll=8)

      def outb(d, carry):
        dv = jnp.full((LANES,), d, jnp.int32)
        rv = plsc.load_gather(rtb, [dv, roff])
        x = corr * rv
        obuf[d, pl.ds((g % 4) * GR + sg * LANES, LANES)] = (
            1.0 / (1.0 + jnp.exp(-x)))
        return carry

      lax.fori_loop(0, EMBED_DIM, outb, 0, unroll=8)

    @pl.when(g + 2 < ngroup)
    def _():
      gstage(g + 2, hbuf, tbuf, gsem, start=True)

    @pl.when(g % 4 == 3)
    def _():
      fb = pl.multiple_of(base + (g - 3) * GR, 128)
      pltpu.sync_copy(obuf, out_t.at[:, pl.ds(fb, 4 * GR)])

  gstage(0, hbuf0, tbuf0, gsem0, start=True)
  gstage(1, hbuf1, tbuf1, gsem1, start=True)

  def gloop(gg, carry):
    garm(2 * gg, hbuf0, tbuf0, gsem0)
    garm(2 * gg + 1, hbuf1, tbuf1, gsem1)
    return carry

  lax.fori_loop(0, ngroup // 2, gloop, 0)


def _build():
  mesh = plsc.VectorSubcoreMesh(core_axis_name="c", subcore_axis_name="s",
                                num_cores=NUM_CORES,
                                num_subcores=NUM_SUBCORES)
  i32, f32 = jnp.int32, jnp.float32
  route = pl.kernel(
      _route_body,
      out_type=(jax.ShapeDtypeStruct((NUM_WORKERS * SLOTS, 128), f32),
                jax.ShapeDtypeStruct((NUM_WORKERS * SLOTS, 128), f32),
                jax.ShapeDtypeStruct((NUM_WORKERS * SLOTS,), i32),
                jax.ShapeDtypeStruct((NUM_WORKERS * SLOTS,), i32)),
      mesh=mesh,
      scratch_types=[
          pltpu.VMEM((STRIP,), i32),
          pltpu.VMEM((WCAP,), i32), pltpu.VMEM((WCAP,), i32),
          pltpu.VMEM((WCAP,), i32), pltpu.VMEM((WCAP,), i32),
          pltpu.VMEM((8 * SCAP,), i32), pltpu.VMEM((8 * SCAP,), i32),
          pltpu.VMEM((8 * SCAP,), i32), pltpu.VMEM((8 * SCAP,), i32),
          pltpu.VMEM((CCAP,), i32), pltpu.VMEM((CCAP,), i32),
          pltpu.VMEM((CCAP,), i32), pltpu.VMEM((CCAP,), i32),
          pltpu.VMEM((CCAP,), i32), pltpu.VMEM((CCAP,), i32),
          pltpu.VMEM((CCAP,), i32), pltpu.VMEM((CCAP,), i32),
          pltpu.VMEM((SLOTS,), i32), pltpu.VMEM((SLOTS,), i32),
          pltpu.VMEM((EMBED_DIM, 512), f32),
          pltpu.VMEM((EMBED_DIM, 512), f32),
          pltpu.VMEM((96, 128), f32),
          pltpu.VMEM((96, 128), f32),
          pltpu.SemaphoreType.DMA, pltpu.SemaphoreType.DMA,
          pltpu.SemaphoreType.DMA, pltpu.SemaphoreType.DMA,
      ],
      compiler_params=pltpu.CompilerParams(needs_layout_passes=False),
  )

  rows_per_worker = BATCH // NUM_WORKERS
  compute = pl.kernel(
      functools.partial(_compute_body, rows_per_worker=rows_per_worker),
      out_type=jax.ShapeDtypeStruct((EMBED_DIM, BATCH), f32),
      mesh=mesh,
      scratch_types=[
          pltpu.VMEM((rows_per_worker,), i32),
          pltpu.VMEM((EMBED_DIM, 1024), f32),
          pltpu.VMEM((rows_per_worker,), i32),
          pltpu.VMEM((rows_per_worker,), i32),
          pltpu.VMEM((2048,), i32), pltpu.VMEM((2048,), i32),
          pltpu.VMEM((2048,), i32), pltpu.VMEM((2048,), i32),
          pltpu.VMEM((32, 128), f32), pltpu.VMEM((32, 128), f32),
          pltpu.VMEM((32, 128), f32), pltpu.VMEM((32, 128), f32),
          pltpu.VMEM((EMBED_DIM, 128), f32),
          pltpu.SemaphoreType.DMA, pltpu.SemaphoreType.DMA,
      ],
      compiler_params=pltpu.CompilerParams(needs_layout_passes=False),
  )
  return route, compute


_route, _compute = _build()


def kernel(head, relation, tail, entity_table, relation_table):
  head = head.astype(jnp.int32)
  relation = relation.astype(jnp.int32)
  tail = tail.astype(jnp.int32)
  etab_t = entity_table.T
  etab_last = jnp.pad(entity_table[999936:], ((0, 64), (0, 0))).T
  rtab = jnp.pad(relation_table, ((0, 1024 - NUM_REL), (0, 0))).T
  hrows, trows, bposh, bpost = _route(head, tail, etab_t, etab_last)
  out_t = _compute(relation, hrows, trows, bposh, bpost, rtab)
  return out_t.T
